# baseline (speedup 1.0000x reference)
import functools

import jax
import jax.numpy as jnp
from jax import lax
from jax.experimental import pallas as pl
from jax.experimental.pallas import tpu as pltpu
from jax.experimental.pallas import tpu_sc as plsc

B = 16384
D = 64
NEG = 20
NC = 2
NS = 16
NW = NC * NS
RPW = B // NW
CHUNK = 32
NCHUNK = RPW // CHUNK
NEG_CHUNK = CHUNK * NEG
NEG_IDX_ROWS = NEG_CHUNK // 128


def _sc_scores(emb, inp, pos, neg2d):
    mesh = plsc.VectorSubcoreMesh(core_axis_name="c", subcore_axis_name="s")

    @functools.partial(
        pl.kernel,
        mesh=mesh,
        compiler_params=pltpu.CompilerParams(
            needs_layout_passes=False, use_tc_tiling_on_sc=False),
        out_type=(
            jax.ShapeDtypeStruct((B,), jnp.float32),
            jax.ShapeDtypeStruct((B * NEG,), jnp.float32),
        ),
        scratch_types=[
            pltpu.VMEM((CHUNK,), jnp.int32),
            pltpu.VMEM((CHUNK,), jnp.int32),
            pltpu.VMEM((NEG_IDX_ROWS, 128), jnp.int32),
            pltpu.VMEM((CHUNK, D), jnp.float32),
            pltpu.VMEM((CHUNK, D), jnp.float32),
            pltpu.VMEM((NEG_CHUNK, D), jnp.float32),
            pltpu.VMEM((CHUNK,), jnp.float32),
            pltpu.VMEM((NEG_CHUNK,), jnp.float32),
            pltpu.SemaphoreType.DMA,
        ],
    )
    def k(emb_h, inp_h, pos_h, neg_h, pos_out, neg_out,
          aidx, pidx, nidx, arows, prows, nrows, postage, negstage, sem):
        wid = lax.axis_index("s") * NC + lax.axis_index("c")
        lane = lax.iota(jnp.int32, 16)

        def chunk_body(c, carry):
            base = wid * RPW + c * CHUNK
            pltpu.sync_copy(inp_h.at[pl.ds(base, CHUNK)], aidx)
            pltpu.sync_copy(pos_h.at[pl.ds(base, CHUNK)], pidx)
            for j in range(NEG_IDX_ROWS):
                pltpu.sync_copy(
                    neg_h.at[pl.ds(base * NEG + j * 128, 128)], nidx.at[j])

            cps = [
                pltpu.async_copy(emb_h.at[aidx], arows, sem),
                pltpu.async_copy(emb_h.at[pidx], prows, sem),
            ]
            for j in range(NEG_IDX_ROWS):
                cps.append(pltpu.async_copy(
                    emb_h.at[nidx.at[j]],
                    nrows.at[pl.ds(j * 128, 128)], sem))
            for cp in cps:
                cp.wait()

            for g in range(CHUNK // 16):
                rowv = lane + g * 16
                nrow_base = rowv * NEG

                def dbody(d, accs):
                    dv = jnp.full((16,), d, dtype=jnp.int32)
                    a_d = plsc.load_gather(arows, [rowv, dv])
                    p_d = plsc.load_gather(prows, [rowv, dv])
                    out = [accs[0] + a_d * p_d]
                    for n in range(NEG):
                        nb = plsc.load_gather(nrows, [nrow_base + n, dv])
                        out.append(accs[n + 1] + a_d * nb)
                    return tuple(out)

                init = tuple(
                    jnp.zeros((16,), jnp.float32) for _ in range(NEG + 1))
                accs = lax.fori_loop(0, D, dbody, init)

                postage[pl.ds(g * 16, 16)] = accs[0]
                for n in range(NEG):
                    negstage[pl.ds(g * NEG * 16 + n * 16, 16)] = accs[n + 1]

            pltpu.sync_copy(postage, pos_out.at[pl.ds(base, CHUNK)])
            pltpu.sync_copy(negstage,
                            neg_out.at[pl.ds(base * NEG, NEG_CHUNK)])
            return carry

        lax.fori_loop(0, NCHUNK, chunk_body, 0)

    return k(emb, inp, pos, neg2d)


def _tc_loss(pos_s, neg_s):
    def body(p_ref, n_ref, o_ref):
        p = jax.nn.log_sigmoid(p_ref[...])
        n = jax.nn.log_sigmoid(n_ref[...])
        o_ref[...] = (jnp.sum(n) - jnp.sum(p))[None, None]

    out = pl.pallas_call(
        body,
        out_shape=jax.ShapeDtypeStruct((1, 1), jnp.float32),
    )(pos_s.reshape(B // 128, 128), neg_s.reshape(B * NEG // 128, 128))
    return out[0, 0]


def kernel(input, pos_input, neg_input, Embedding):
    inp = input.astype(jnp.int32)
    pos = pos_input.astype(jnp.int32)
    neg = neg_input.astype(jnp.int32).reshape(B * NEG)
    pos_s, neg_s = _sc_scores(Embedding, inp, pos, neg)
    return _tc_loss(pos_s, neg_s)


# speedup vs baseline: 1.0759x; 1.0759x over previous
import functools

import jax
import jax.numpy as jnp
from jax import lax
from jax.experimental import pallas as pl
from jax.experimental.pallas import tpu as pltpu
from jax.experimental.pallas import tpu_sc as plsc

B = 16384
D = 64
NEG = 20
NC = 2
NS = 16
NW = NC * NS
RPW = B // NW
CHUNK = 32
NCHUNK = RPW // CHUNK
NEG_CHUNK = CHUNK * NEG
NEG_IDX_ROWS = NEG_CHUNK // 128
AP_IDX_ROWS = RPW // 128


def _sc_scores(emb, inp2d, pos2d, neg2d):
    mesh = plsc.VectorSubcoreMesh(core_axis_name="c", subcore_axis_name="s")

    @functools.partial(
        pl.kernel,
        mesh=mesh,
        compiler_params=pltpu.CompilerParams(
            needs_layout_passes=False, use_tc_tiling_on_sc=False),
        out_type=(
            jax.ShapeDtypeStruct((B,), jnp.float32),
            jax.ShapeDtypeStruct((B * NEG,), jnp.float32),
        ),
        scratch_types=[
            pltpu.VMEM((AP_IDX_ROWS, 128), jnp.int32),
            pltpu.VMEM((AP_IDX_ROWS, 128), jnp.int32),
            pltpu.VMEM((RPW // CHUNK * NEG_IDX_ROWS, 128), jnp.int32),
            pltpu.VMEM((RPW, D), jnp.float32),
            pltpu.VMEM((NEG_CHUNK, D), jnp.float32),
            pltpu.VMEM((NEG_CHUNK, D), jnp.float32),
            pltpu.VMEM((RPW,), jnp.float32),
            pltpu.VMEM((NEG_CHUNK,), jnp.float32),
            pltpu.SemaphoreType.DMA,
            pltpu.SemaphoreType.DMA,
            pltpu.SemaphoreType.DMA,
        ],
    )
    def k(emb_h, inp_h, pos_h, neg_h, pos_out, neg_out,
          aidx, pidx, nidx, abuf, nbufA, nbufB, postage, negstage,
          semP, semA, semB):
        wid = lax.axis_index("s") * NC + lax.axis_index("c")
        lane = lax.iota(jnp.int32, 16)

        pltpu.sync_copy(inp_h.at[pl.ds(wid * AP_IDX_ROWS, AP_IDX_ROWS)], aidx)
        pltpu.sync_copy(pos_h.at[pl.ds(wid * AP_IDX_ROWS, AP_IDX_ROWS)], pidx)
        nidx_rows = NCHUNK * NEG_IDX_ROWS
        pltpu.sync_copy(neg_h.at[pl.ds(wid * nidx_rows, nidx_rows)], nidx)

        def fire_neg(c, buf, sem):
            for j in range(NEG_IDX_ROWS):
                pltpu.async_copy(
                    emb_h.at[nidx.at[c * NEG_IDX_ROWS + j]],
                    buf.at[pl.ds(j * 128, 128)], sem)

        def drain(buf, sem):
            pltpu.make_async_copy(
                emb_h.at[pl.ds(0, buf.shape[0])], buf, sem).wait()

        for j in range(AP_IDX_ROWS):
            pltpu.async_copy(emb_h.at[aidx.at[j]],
                             abuf.at[pl.ds(j * 128, 128)], semP)
            pltpu.async_copy(emb_h.at[pidx.at[j]],
                             nbufA.at[pl.ds(j * 128, 128)], semP)
        fire_neg(0, nbufB, semB)
        pltpu.make_async_copy(
            emb_h.at[pl.ds(0, RPW)], abuf, semP).wait()
        pltpu.make_async_copy(
            emb_h.at[pl.ds(0, RPW)], nbufA.at[pl.ds(0, RPW)], semP).wait()

        def pos_group(g, carry):
            rowv = lane + g * 16

            def dbody(d, acc):
                dv = jnp.full((16,), d, dtype=jnp.int32)
                a_d = plsc.load_gather(abuf, [rowv, dv])
                p_d = plsc.load_gather(nbufA, [rowv, dv])
                return acc + a_d * p_d

            acc = lax.fori_loop(0, D, dbody, jnp.zeros((16,), jnp.float32))
            postage[pl.ds(g * 16, 16)] = acc
            return carry

        lax.fori_loop(0, RPW // 16, pos_group, 0)
        pltpu.sync_copy(postage, pos_out.at[pl.ds(wid * RPW, RPW)])
        fire_neg(1, nbufA, semA)

        def neg_chunk(c, buf):
            for g in range(CHUNK // 16):
                rowv = lane + g * 16
                arow = rowv + c * CHUNK
                nrow_base = rowv * NEG

                def dbody(d, accs):
                    dv = jnp.full((16,), d, dtype=jnp.int32)
                    a_d = plsc.load_gather(abuf, [arow, dv])
                    out = []
                    for n in range(NEG):
                        nb = plsc.load_gather(buf, [nrow_base + n, dv])
                        out.append(accs[n] + a_d * nb)
                    return tuple(out)

                init = tuple(
                    jnp.zeros((16,), jnp.float32) for _ in range(NEG))
                accs = lax.fori_loop(0, D, dbody, init)
                for n in range(NEG):
                    negstage[pl.ds(g * NEG * 16 + n * 16, 16)] = accs[n]
            pltpu.sync_copy(
                negstage,
                neg_out.at[pl.ds((wid * NCHUNK + c) * NEG_CHUNK, NEG_CHUNK)])

        def pair_body(i, carry):
            c_even = i * 2
            drain(nbufB, semB)
            neg_chunk(c_even, nbufB)

            @pl.when(i < NCHUNK // 2 - 1)
            def _():
                fire_neg(c_even + 2, nbufB, semB)

            drain(nbufA, semA)
            neg_chunk(c_even + 1, nbufA)

            @pl.when(i < NCHUNK // 2 - 1)
            def _():
                fire_neg(c_even + 3, nbufA, semA)

            return carry

        lax.fori_loop(0, NCHUNK // 2, pair_body, 0)

    return k(emb, inp2d, pos2d, neg2d)


def _tc_loss(pos_s, neg_s):
    def body(p_ref, n_ref, o_ref):
        p = jax.nn.log_sigmoid(p_ref[...])
        n = jax.nn.log_sigmoid(n_ref[...])
        o_ref[...] = (jnp.sum(n) - jnp.sum(p))[None, None]

    out = pl.pallas_call(
        body,
        out_shape=jax.ShapeDtypeStruct((1, 1), jnp.float32),
    )(pos_s.reshape(B // 128, 128), neg_s.reshape(B * NEG // 128, 128))
    return out[0, 0]


def kernel(input, pos_input, neg_input, Embedding):
    inp = input.astype(jnp.int32).reshape(B // 128, 128)
    pos = pos_input.astype(jnp.int32).reshape(B // 128, 128)
    neg = neg_input.astype(jnp.int32).reshape(B * NEG // 128, 128)
    pos_s, neg_s = _sc_scores(Embedding, inp, pos, neg)
    return _tc_loss(pos_s, neg_s)


# speedup vs baseline: 1.2366x; 1.1493x over previous
import functools

import jax
import jax.numpy as jnp
from jax import lax
from jax.experimental import pallas as pl
from jax.experimental.pallas import tpu as pltpu
from jax.experimental.pallas import tpu_sc as plsc

B = 16384
V = 1000000
D = 64
NEG = 20
NC = 2
NS = 16
NW = NC * NS
RPW = B // NW
CHUNK = 128
NCHUNK = RPW // CHUNK
TBLK = 1024
NTB = (V + TBLK - 1) // TBLK
VPAD = NTB * TBLK


def _widen_table(emb):
    def body(x_ref, o_ref):
        o_ref[:, pl.ds(0, D)] = x_ref[...].T

    return pl.pallas_call(
        body,
        grid=(NTB,),
        in_specs=[pl.BlockSpec((D, TBLK), lambda g: (0, g))],
        out_specs=pl.BlockSpec((TBLK, 128), lambda g: (g, 0)),
        out_shape=jax.ShapeDtypeStruct((VPAD, 128), jnp.float32),
    )(emb.T)


def _sc_scores(ew, inp2d, pos2d, negT2d):
    mesh = plsc.VectorSubcoreMesh(core_axis_name="c", subcore_axis_name="s")

    @functools.partial(
        pl.kernel,
        mesh=mesh,
        compiler_params=pltpu.CompilerParams(
            needs_layout_passes=False, use_tc_tiling_on_sc=True),
        out_type=(
            jax.ShapeDtypeStruct((B,), jnp.float32),
            jax.ShapeDtypeStruct((B * NEG,), jnp.float32),
        ),
        scratch_types=[
            pltpu.VMEM((NCHUNK, 128), jnp.int32),
            pltpu.VMEM((NCHUNK, 128), jnp.int32),
            pltpu.VMEM((NEG * NCHUNK, 128), jnp.int32),
            pltpu.VMEM((CHUNK, 128), jnp.float32),
            pltpu.VMEM((CHUNK, 128), jnp.float32),
            pltpu.VMEM((CHUNK, 128), jnp.float32),
            pltpu.VMEM((CHUNK, 128), jnp.float32),
            pltpu.VMEM((CHUNK, 128), jnp.float32),
            pltpu.VMEM((CHUNK, 128), jnp.float32),
            pltpu.VMEM((CHUNK, 128), jnp.float32),
            pltpu.VMEM((CHUNK,), jnp.float32),
            pltpu.VMEM((CHUNK * NEG,), jnp.float32),
            pltpu.SemaphoreType.DMA,
            pltpu.SemaphoreType.DMA,
            pltpu.SemaphoreType.DMA,
            pltpu.SemaphoreType.DMA,
            pltpu.SemaphoreType.DMA,
        ],
    )
    def k(ew_h, inp_h, pos_h, neg_h, pos_out, neg_out,
          aidx, pidx, nidx, waA, waB, wpA, wpB, wnA, wnB, wnC,
          postage, negstage, semA, semP, semN0, semN1, semN2):
        wid = lax.axis_index("s") * NC + lax.axis_index("c")
        lane = lax.iota(jnp.int32, 16)
        wa = (waA, waB)
        wp = (wpA, wpB)
        wn = (wnA, wnB, wnC)
        semN = (semN0, semN1, semN2)

        pltpu.sync_copy(inp_h.at[pl.ds(wid * NCHUNK, NCHUNK)], aidx)
        pltpu.sync_copy(pos_h.at[pl.ds(wid * NCHUNK, NCHUNK)], pidx)
        for n in range(NEG):
            pltpu.sync_copy(
                neg_h.at[pl.ds(n * (B // 128) + wid * NCHUNK, NCHUNK)],
                nidx.at[pl.ds(n * NCHUNK, NCHUNK)])

        def fire_ap(c):
            pltpu.async_copy(ew_h.at[aidx.at[c]], wa[c % 2], semA)
            pltpu.async_copy(ew_h.at[pidx.at[c]], wp[c % 2], semP)

        def fire_n(c, n):
            bi = (c * NEG + n) % 3
            pltpu.async_copy(ew_h.at[nidx.at[n * NCHUNK + c]],
                             wn[bi], semN[bi])

        def drain(buf, sem):
            pltpu.make_async_copy(ew_h.at[pl.ds(0, CHUNK)], buf, sem).wait()

        fire_ap(0)
        fire_n(0, 0)
        fire_n(0, 1)

        def dots(bufx, bufy, stage, soff):
            def group(g, carry):
                rowv = lane + g * 16

                def dbody(d, acc):
                    rot = (lane + d) & (D - 1)
                    x_d = plsc.load_gather(bufx, [rowv, rot])
                    y_d = plsc.load_gather(bufy, [rowv, rot])
                    return acc + x_d * y_d

                acc = lax.fori_loop(0, D, dbody,
                                    jnp.zeros((16,), jnp.float32))
                stage[pl.ds(soff + g * 16, 16)] = acc
                return carry

            lax.fori_loop(0, CHUNK // 16, group, 0)

        for c in range(NCHUNK):
            gc = wid * NCHUNK + c
            drain(wa[c % 2], semA)
            drain(wp[c % 2], semP)
            if c + 1 < NCHUNK:
                fire_ap(c + 1)
            dots(wa[c % 2], wp[c % 2], postage, 0)
            pltpu.sync_copy(postage, pos_out.at[pl.ds(gc * CHUNK, CHUNK)])
            for n in range(NEG):
                bi = (c * NEG + n) % 3
                drain(wn[bi], semN[bi])
                nf = n + 2
                if nf < NEG:
                    fire_n(c, nf)
                elif c + 1 < NCHUNK:
                    fire_n(c + 1, nf - NEG)
                dots(wa[c % 2], wn[bi], negstage, n * CHUNK)
            pltpu.sync_copy(
                negstage,
                neg_out.at[pl.ds(gc * CHUNK * NEG, CHUNK * NEG)])

    return k(ew, inp2d, pos2d, negT2d)


def _tc_loss(pos_s, neg_s):
    def body(p_ref, n_ref, o_ref):
        p = jax.nn.log_sigmoid(p_ref[...])
        n = jax.nn.log_sigmoid(n_ref[...])
        o_ref[...] = (jnp.sum(n) - jnp.sum(p))[None, None]

    out = pl.pallas_call(
        body,
        out_shape=jax.ShapeDtypeStruct((1, 1), jnp.float32),
    )(pos_s.reshape(B // 128, 128), neg_s.reshape(B * NEG // 128, 128))
    return out[0, 0]


def kernel(input, pos_input, neg_input, Embedding):
    inp = input.astype(jnp.int32).reshape(B // 128, 128)
    pos = pos_input.astype(jnp.int32).reshape(B // 128, 128)
    negT = neg_input.astype(jnp.int32).T.reshape(NEG * B // 128, 128)
    ew = _widen_table(Embedding)
    pos_s, neg_s = _sc_scores(ew, inp, pos, negT)
    return _tc_loss(pos_s, neg_s)


# speedup vs baseline: 1.6898x; 1.3665x over previous
import functools

import jax
import jax.numpy as jnp
from jax import lax
from jax.experimental import pallas as pl
from jax.experimental.pallas import tpu as pltpu
from jax.experimental.pallas import tpu_sc as plsc

B = 16384
V = 1000000
D = 64
NEG = 20
NC = 2
NS = 16
NW = NC * NS
RPW = B // NW
CHUNK = 128
NCHUNK = RPW // CHUNK
TBLK = 2048
NTB = (V + TBLK - 1) // TBLK
VPAD = NTB * TBLK


def _widen_table(emb):
    def body(x_ref, o_ref):
        o_ref[:, pl.ds(0, D)] = x_ref[...].T

    return pl.pallas_call(
        body,
        grid=(NTB,),
        in_specs=[pl.BlockSpec((D, TBLK), lambda g: (0, g))],
        out_specs=pl.BlockSpec((TBLK, 128), lambda g: (g, 0)),
        out_shape=jax.ShapeDtypeStruct((VPAD, 128), jnp.float32),
    )(emb.T)


def _sc_scores(ew, inp2d, pos2d, negT2d):
    mesh = plsc.VectorSubcoreMesh(core_axis_name="c", subcore_axis_name="s")

    @functools.partial(
        pl.kernel,
        mesh=mesh,
        compiler_params=pltpu.CompilerParams(
            needs_layout_passes=False, use_tc_tiling_on_sc=True),
        out_type=(
            jax.ShapeDtypeStruct((B,), jnp.float32),
            jax.ShapeDtypeStruct((B * NEG,), jnp.float32),
        ),
        scratch_types=[
            pltpu.VMEM((NCHUNK, 128), jnp.int32),
            pltpu.VMEM((NCHUNK, 128), jnp.int32),
            pltpu.VMEM((NEG * NCHUNK, 128), jnp.int32),
            pltpu.VMEM((CHUNK, 128), jnp.float32),
            pltpu.VMEM((CHUNK, 128), jnp.float32),
            pltpu.VMEM((CHUNK, 128), jnp.float32),
            pltpu.VMEM((CHUNK, 128), jnp.float32),
            pltpu.VMEM((CHUNK, 128), jnp.float32),
            pltpu.VMEM((CHUNK, 128), jnp.float32),
            pltpu.VMEM((CHUNK, 128), jnp.float32),
            pltpu.VMEM((CHUNK,), jnp.float32),
            pltpu.VMEM((CHUNK * NEG,), jnp.float32),
            pltpu.SemaphoreType.DMA,
            pltpu.SemaphoreType.DMA,
            pltpu.SemaphoreType.DMA,
            pltpu.SemaphoreType.DMA,
            pltpu.SemaphoreType.DMA,
        ],
    )
    def k(ew_h, inp_h, pos_h, neg_h, pos_out, neg_out,
          aidx, pidx, nidx, waA, waB, wpA, wpB, wnA, wnB, wnC,
          postage, negstage, semA, semP, semN0, semN1, semN2):
        wid = lax.axis_index("s") * NC + lax.axis_index("c")
        lane = lax.iota(jnp.int32, 16)
        wa = (waA, waB)
        wp = (wpA, wpB)
        wn = (wnA, wnB, wnC)
        semN = (semN0, semN1, semN2)

        pltpu.sync_copy(inp_h.at[pl.ds(wid * NCHUNK, NCHUNK)], aidx)
        pltpu.sync_copy(pos_h.at[pl.ds(wid * NCHUNK, NCHUNK)], pidx)
        for n in range(NEG):
            pltpu.sync_copy(
                neg_h.at[pl.ds(n * (B // 128) + wid * NCHUNK, NCHUNK)],
                nidx.at[pl.ds(n * NCHUNK, NCHUNK)])

        def fire_ap(c):
            pltpu.async_copy(ew_h.at[aidx.at[c]], wa[c % 2], semA)
            pltpu.async_copy(ew_h.at[pidx.at[c]], wp[c % 2], semP)

        def fire_n(c, n):
            bi = (c * NEG + n) % 3
            pltpu.async_copy(ew_h.at[nidx.at[n * NCHUNK + c]],
                             wn[bi], semN[bi])

        def drain(buf, sem):
            pltpu.make_async_copy(ew_h.at[pl.ds(0, CHUNK)], buf, sem).wait()

        fire_ap(0)
        fire_n(0, 0)
        fire_n(0, 1)

        def dots(bufx, bufy, stage, soff):
            def group(g, carry):
                rowv = lane + g * 16

                def dbody(d, acc):
                    rot = (lane + d) & (D - 1)
                    x_d = plsc.load_gather(bufx, [rowv, rot])
                    y_d = plsc.load_gather(bufy, [rowv, rot])
                    return acc + x_d * y_d

                acc = lax.fori_loop(0, D, dbody,
                                    jnp.zeros((16,), jnp.float32))
                stage[pl.ds(soff + g * 16, 16)] = acc
                return carry

            lax.fori_loop(0, CHUNK // 16, group, 0)

        for c in range(NCHUNK):
            gc = wid * NCHUNK + c
            drain(wa[c % 2], semA)
            drain(wp[c % 2], semP)
            if c + 1 < NCHUNK:
                fire_ap(c + 1)
            dots(wa[c % 2], wp[c % 2], postage, 0)
            pltpu.sync_copy(postage, pos_out.at[pl.ds(gc * CHUNK, CHUNK)])
            for n in range(NEG):
                bi = (c * NEG + n) % 3
                drain(wn[bi], semN[bi])
                nf = n + 2
                if nf < NEG:
                    fire_n(c, nf)
                elif c + 1 < NCHUNK:
                    fire_n(c + 1, nf - NEG)
                dots(wa[c % 2], wn[bi], negstage, n * CHUNK)
            pltpu.sync_copy(
                negstage,
                neg_out.at[pl.ds(gc * CHUNK * NEG, CHUNK * NEG)])

    return k(ew, inp2d, pos2d, negT2d)


def _tc_loss(pos_s, neg_s):
    def body(p_ref, n_ref, o_ref):
        p = jax.nn.log_sigmoid(p_ref[...])
        n = jax.nn.log_sigmoid(n_ref[...])
        o_ref[...] = (jnp.sum(n) - jnp.sum(p))[None, None]

    out = pl.pallas_call(
        body,
        out_shape=jax.ShapeDtypeStruct((1, 1), jnp.float32),
    )(pos_s.reshape(B // 128, 128), neg_s.reshape(B * NEG // 128, 128))
    return out[0, 0]


def kernel(input, pos_input, neg_input, Embedding):
    inp = input.astype(jnp.int32).reshape(B // 128, 128)
    pos = pos_input.astype(jnp.int32).reshape(B // 128, 128)
    negT = neg_input.astype(jnp.int32).T.reshape(NEG * B // 128, 128)
    ew = _widen_table(Embedding)
    pos_s, neg_s = _sc_scores(ew, inp, pos, negT)
    return _tc_loss(pos_s, neg_s)


# speedup vs baseline: 2.3677x; 1.4012x over previous
import functools

import jax
import jax.numpy as jnp
from jax import lax
from jax.experimental import pallas as pl
from jax.experimental.pallas import tpu as pltpu
from jax.experimental.pallas import tpu_sc as plsc

B = 16384
V = 1000000
D = 64
NEG = 20
NC = 2
NS = 16
NW = NC * NS
RPW = B // NW
CHUNK = 128
NCHUNK = RPW // CHUNK
TBLK = 8192
NTB = (V + TBLK - 1) // TBLK
VPAD = NTB * TBLK


def _widen_table(emb):
    def body(x_ref, o_ref):
        o_ref[:, pl.ds(0, D)] = x_ref[...].T

    return pl.pallas_call(
        body,
        grid=(NTB,),
        in_specs=[pl.BlockSpec((D, TBLK), lambda g: (0, g))],
        out_specs=pl.BlockSpec((TBLK, 128), lambda g: (g, 0)),
        out_shape=jax.ShapeDtypeStruct((VPAD, 128), jnp.float32),
    )(emb.T)


def _sc_scores(ew, inp2d, pos2d, negT2d):
    mesh = plsc.VectorSubcoreMesh(core_axis_name="c", subcore_axis_name="s")

    @functools.partial(
        pl.kernel,
        mesh=mesh,
        compiler_params=pltpu.CompilerParams(
            needs_layout_passes=False, use_tc_tiling_on_sc=True),
        out_type=(
            jax.ShapeDtypeStruct((B,), jnp.float32),
            jax.ShapeDtypeStruct((B * NEG,), jnp.float32),
        ),
        scratch_types=[
            pltpu.VMEM((NCHUNK, 128), jnp.int32),
            pltpu.VMEM((NCHUNK, 128), jnp.int32),
            pltpu.VMEM((NEG * NCHUNK, 128), jnp.int32),
            pltpu.VMEM((CHUNK, 128), jnp.float32),
            pltpu.VMEM((CHUNK, 128), jnp.float32),
            pltpu.VMEM((CHUNK, 128), jnp.float32),
            pltpu.VMEM((CHUNK, 128), jnp.float32),
            pltpu.VMEM((CHUNK, 128), jnp.float32),
            pltpu.VMEM((CHUNK, 128), jnp.float32),
            pltpu.VMEM((CHUNK, 128), jnp.float32),
            pltpu.VMEM((CHUNK,), jnp.float32),
            pltpu.VMEM((CHUNK * NEG,), jnp.float32),
            pltpu.SemaphoreType.DMA,
            pltpu.SemaphoreType.DMA,
            pltpu.SemaphoreType.DMA,
            pltpu.SemaphoreType.DMA,
            pltpu.SemaphoreType.DMA,
        ],
    )
    def k(ew_h, inp_h, pos_h, neg_h, pos_out, neg_out,
          aidx, pidx, nidx, waA, waB, wpA, wpB, wnA, wnB, wnC,
          postage, negstage, semA, semP, semN0, semN1, semN2):
        wid = lax.axis_index("s") * NC + lax.axis_index("c")
        lane = lax.iota(jnp.int32, 16)
        wa = (waA, waB)
        wp = (wpA, wpB)
        wn = (wnA, wnB, wnC)
        semN = (semN0, semN1, semN2)

        pltpu.sync_copy(inp_h.at[pl.ds(wid * NCHUNK, NCHUNK)], aidx)
        pltpu.sync_copy(pos_h.at[pl.ds(wid * NCHUNK, NCHUNK)], pidx)
        for n in range(NEG):
            pltpu.sync_copy(
                neg_h.at[pl.ds(n * (B // 128) + wid * NCHUNK, NCHUNK)],
                nidx.at[pl.ds(n * NCHUNK, NCHUNK)])

        def fire_ap(c):
            pltpu.async_copy(ew_h.at[aidx.at[c]], wa[c % 2], semA)
            pltpu.async_copy(ew_h.at[pidx.at[c]], wp[c % 2], semP)

        def fire_n(c, n):
            bi = (c * NEG + n) % 3
            pltpu.async_copy(ew_h.at[nidx.at[n * NCHUNK + c]],
                             wn[bi], semN[bi])

        def drain(buf, sem):
            pltpu.make_async_copy(ew_h.at[pl.ds(0, CHUNK)], buf, sem).wait()

        fire_ap(0)
        fire_n(0, 0)
        fire_n(0, 1)

        def dots(bufx, bufy, stage, soff):
            def group(g, carry):
                rowv = lane + g * 16

                def dbody(d, acc):
                    rot = (lane + d) & (D - 1)
                    x_d = plsc.load_gather(bufx, [rowv, rot])
                    y_d = plsc.load_gather(bufy, [rowv, rot])
                    return acc + x_d * y_d

                acc = lax.fori_loop(0, D, dbody,
                                    jnp.zeros((16,), jnp.float32))
                stage[pl.ds(soff + g * 16, 16)] = acc
                return carry

            lax.fori_loop(0, CHUNK // 16, group, 0)

        for c in range(NCHUNK):
            gc = wid * NCHUNK + c
            drain(wa[c % 2], semA)
            drain(wp[c % 2], semP)
            if c + 1 < NCHUNK:
                fire_ap(c + 1)
            dots(wa[c % 2], wp[c % 2], postage, 0)
            pltpu.sync_copy(postage, pos_out.at[pl.ds(gc * CHUNK, CHUNK)])
            for n in range(NEG):
                bi = (c * NEG + n) % 3
                drain(wn[bi], semN[bi])
                nf = n + 2
                if nf < NEG:
                    fire_n(c, nf)
                elif c + 1 < NCHUNK:
                    fire_n(c + 1, nf - NEG)
                dots(wa[c % 2], wn[bi], negstage, n * CHUNK)
            pltpu.sync_copy(
                negstage,
                neg_out.at[pl.ds(gc * CHUNK * NEG, CHUNK * NEG)])

    return k(ew, inp2d, pos2d, negT2d)


def _tc_loss(pos_s, neg_s):
    def body(p_ref, n_ref, o_ref):
        p = jax.nn.log_sigmoid(p_ref[...])
        n = jax.nn.log_sigmoid(n_ref[...])
        o_ref[...] = (jnp.sum(n) - jnp.sum(p))[None, None]

    out = pl.pallas_call(
        body,
        out_shape=jax.ShapeDtypeStruct((1, 1), jnp.float32),
    )(pos_s.reshape(B // 128, 128), neg_s.reshape(B * NEG // 128, 128))
    return out[0, 0]


def kernel(input, pos_input, neg_input, Embedding):
    inp = input.astype(jnp.int32).reshape(B // 128, 128)
    pos = pos_input.astype(jnp.int32).reshape(B // 128, 128)
    negT = neg_input.astype(jnp.int32).T.reshape(NEG * B // 128, 128)
    ew = _widen_table(Embedding)
    pos_s, neg_s = _sc_scores(ew, inp, pos, negT)
    return _tc_loss(pos_s, neg_s)


# speedup vs baseline: 2.4747x; 1.0452x over previous
import functools

import jax
import jax.numpy as jnp
from jax import lax
from jax.experimental import pallas as pl
from jax.experimental.pallas import tpu as pltpu
from jax.experimental.pallas import tpu_sc as plsc

B = 16384
V = 1000000
D = 64
NEG = 20
NC = 2
NS = 16
NW = NC * NS
RPW = B // NW
CHUNK = 128
NCHUNK = RPW // CHUNK
TBLK = 16384
NTB = (V + TBLK - 1) // TBLK
VPAD = NTB * TBLK


def _widen_table(emb):
    def body(x_ref, o_ref):
        o_ref[:, pl.ds(0, D)] = x_ref[...].T

    return pl.pallas_call(
        body,
        grid=(NTB,),
        in_specs=[pl.BlockSpec((D, TBLK), lambda g: (0, g))],
        out_specs=pl.BlockSpec((TBLK, 128), lambda g: (g, 0)),
        out_shape=jax.ShapeDtypeStruct((VPAD, 128), jnp.float32),
    )(emb.T)


def _sc_scores(ew, inp2d, pos2d, negT2d):
    mesh = plsc.VectorSubcoreMesh(core_axis_name="c", subcore_axis_name="s")

    @functools.partial(
        pl.kernel,
        mesh=mesh,
        compiler_params=pltpu.CompilerParams(
            needs_layout_passes=False, use_tc_tiling_on_sc=True),
        out_type=(
            jax.ShapeDtypeStruct((B,), jnp.float32),
            jax.ShapeDtypeStruct((B * NEG,), jnp.float32),
        ),
        scratch_types=[
            pltpu.VMEM((NCHUNK, 128), jnp.int32),
            pltpu.VMEM((NCHUNK, 128), jnp.int32),
            pltpu.VMEM((NEG * NCHUNK, 128), jnp.int32),
            pltpu.VMEM((CHUNK, 128), jnp.float32),
            pltpu.VMEM((CHUNK, 128), jnp.float32),
            pltpu.VMEM((CHUNK, 128), jnp.float32),
            pltpu.VMEM((CHUNK, 128), jnp.float32),
            pltpu.VMEM((CHUNK, 128), jnp.float32),
            pltpu.VMEM((CHUNK, 128), jnp.float32),
            pltpu.VMEM((CHUNK, 128), jnp.float32),
            pltpu.VMEM((CHUNK,), jnp.float32),
            pltpu.VMEM((CHUNK * NEG,), jnp.float32),
            pltpu.SemaphoreType.DMA,
            pltpu.SemaphoreType.DMA,
            pltpu.SemaphoreType.DMA,
            pltpu.SemaphoreType.DMA,
            pltpu.SemaphoreType.DMA,
        ],
    )
    def k(ew_h, inp_h, pos_h, neg_h, pos_out, neg_out,
          aidx, pidx, nidx, waA, waB, wpA, wpB, wnA, wnB, wnC,
          postage, negstage, semA, semP, semN0, semN1, semN2):
        wid = lax.axis_index("s") * NC + lax.axis_index("c")
        lane = lax.iota(jnp.int32, 16)
        wa = (waA, waB)
        wp = (wpA, wpB)
        wn = (wnA, wnB, wnC)
        semN = (semN0, semN1, semN2)

        pltpu.sync_copy(inp_h.at[pl.ds(wid * NCHUNK, NCHUNK)], aidx)
        pltpu.sync_copy(pos_h.at[pl.ds(wid * NCHUNK, NCHUNK)], pidx)
        for n in range(NEG):
            pltpu.sync_copy(
                neg_h.at[pl.ds(n * (B // 128) + wid * NCHUNK, NCHUNK)],
                nidx.at[pl.ds(n * NCHUNK, NCHUNK)])

        def fire_ap(c):
            pltpu.async_copy(ew_h.at[aidx.at[c]], wa[c % 2], semA)
            pltpu.async_copy(ew_h.at[pidx.at[c]], wp[c % 2], semP)

        def fire_n(c, n):
            bi = (c * NEG + n) % 3
            pltpu.async_copy(ew_h.at[nidx.at[n * NCHUNK + c]],
                             wn[bi], semN[bi])

        def drain(buf, sem):
            pltpu.make_async_copy(ew_h.at[pl.ds(0, CHUNK)], buf, sem).wait()

        fire_ap(0)
        fire_n(0, 0)
        fire_n(0, 1)

        def dots(bufx, bufy, stage, soff):
            def group(g, carry):
                rowv = lane + g * 16

                def dbody(d, acc):
                    rot = (lane + d) & (D - 1)
                    x_d = plsc.load_gather(bufx, [rowv, rot])
                    y_d = plsc.load_gather(bufy, [rowv, rot])
                    return acc + x_d * y_d

                acc = lax.fori_loop(0, D, dbody,
                                    jnp.zeros((16,), jnp.float32))
                stage[pl.ds(soff + g * 16, 16)] = acc
                return carry

            lax.fori_loop(0, CHUNK // 16, group, 0)

        for c in range(NCHUNK):
            gc = wid * NCHUNK + c
            drain(wa[c % 2], semA)
            drain(wp[c % 2], semP)
            if c + 1 < NCHUNK:
                fire_ap(c + 1)
            dots(wa[c % 2], wp[c % 2], postage, 0)
            pltpu.sync_copy(postage, pos_out.at[pl.ds(gc * CHUNK, CHUNK)])
            for n in range(NEG):
                bi = (c * NEG + n) % 3
                drain(wn[bi], semN[bi])
                nf = n + 2
                if nf < NEG:
                    fire_n(c, nf)
                elif c + 1 < NCHUNK:
                    fire_n(c + 1, nf - NEG)
                dots(wa[c % 2], wn[bi], negstage, n * CHUNK)
            pltpu.sync_copy(
                negstage,
                neg_out.at[pl.ds(gc * CHUNK * NEG, CHUNK * NEG)])

    return k(ew, inp2d, pos2d, negT2d)


def _tc_loss(pos_s, neg_s):
    def body(p_ref, n_ref, o_ref):
        p = jax.nn.log_sigmoid(p_ref[...])
        n = jax.nn.log_sigmoid(n_ref[...])
        o_ref[...] = (jnp.sum(n) - jnp.sum(p))[None, None]

    out = pl.pallas_call(
        body,
        out_shape=jax.ShapeDtypeStruct((1, 1), jnp.float32),
    )(pos_s.reshape(B // 128, 128), neg_s.reshape(B * NEG // 128, 128))
    return out[0, 0]


def kernel(input, pos_input, neg_input, Embedding):
    inp = input.astype(jnp.int32).reshape(B // 128, 128)
    pos = pos_input.astype(jnp.int32).reshape(B // 128, 128)
    negT = neg_input.astype(jnp.int32).T.reshape(NEG * B // 128, 128)
    ew = _widen_table(Embedding)
    pos_s, neg_s = _sc_scores(ew, inp, pos, negT)
    return _tc_loss(pos_s, neg_s)


# speedup vs baseline: 2.5055x; 1.0124x over previous
import functools

import jax
import jax.numpy as jnp
from jax import lax
from jax.experimental import pallas as pl
from jax.experimental.pallas import tpu as pltpu
from jax.experimental.pallas import tpu_sc as plsc

B = 16384
V = 1000000
D = 64
NEG = 20
NC = 2
NS = 16
NW = NC * NS
RPW = B // NW
CHUNK = 128
NCHUNK = RPW // CHUNK
TBLK = 32768
NTB = (V + TBLK - 1) // TBLK
VPAD = NTB * TBLK


def _widen_table(emb):
    def body(x_ref, o_ref):
        o_ref[:, pl.ds(0, D)] = x_ref[...].T

    return pl.pallas_call(
        body,
        grid=(NTB,),
        in_specs=[pl.BlockSpec((D, TBLK), lambda g: (0, g))],
        out_specs=pl.BlockSpec((TBLK, 128), lambda g: (g, 0)),
        out_shape=jax.ShapeDtypeStruct((VPAD, 128), jnp.float32),
    )(emb.T)


def _sc_scores(ew, inp2d, pos2d, negT2d):
    mesh = plsc.VectorSubcoreMesh(core_axis_name="c", subcore_axis_name="s")

    @functools.partial(
        pl.kernel,
        mesh=mesh,
        compiler_params=pltpu.CompilerParams(
            needs_layout_passes=False, use_tc_tiling_on_sc=True),
        out_type=(
            jax.ShapeDtypeStruct((B,), jnp.float32),
            jax.ShapeDtypeStruct((B * NEG,), jnp.float32),
        ),
        scratch_types=[
            pltpu.VMEM((NCHUNK, 128), jnp.int32),
            pltpu.VMEM((NCHUNK, 128), jnp.int32),
            pltpu.VMEM((NEG * NCHUNK, 128), jnp.int32),
            pltpu.VMEM((CHUNK, 128), jnp.float32),
            pltpu.VMEM((CHUNK, 128), jnp.float32),
            pltpu.VMEM((CHUNK, 128), jnp.float32),
            pltpu.VMEM((CHUNK, 128), jnp.float32),
            pltpu.VMEM((CHUNK, 128), jnp.float32),
            pltpu.VMEM((CHUNK, 128), jnp.float32),
            pltpu.VMEM((CHUNK, 128), jnp.float32),
            pltpu.VMEM((CHUNK,), jnp.float32),
            pltpu.VMEM((CHUNK * NEG,), jnp.float32),
            pltpu.SemaphoreType.DMA,
            pltpu.SemaphoreType.DMA,
            pltpu.SemaphoreType.DMA,
            pltpu.SemaphoreType.DMA,
            pltpu.SemaphoreType.DMA,
        ],
    )
    def k(ew_h, inp_h, pos_h, neg_h, pos_out, neg_out,
          aidx, pidx, nidx, waA, waB, wpA, wpB, wnA, wnB, wnC,
          postage, negstage, semA, semP, semN0, semN1, semN2):
        wid = lax.axis_index("s") * NC + lax.axis_index("c")
        lane = lax.iota(jnp.int32, 16)
        wa = (waA, waB)
        wp = (wpA, wpB)
        wn = (wnA, wnB, wnC)
        semN = (semN0, semN1, semN2)

        pltpu.sync_copy(inp_h.at[pl.ds(wid * NCHUNK, NCHUNK)], aidx)
        pltpu.sync_copy(pos_h.at[pl.ds(wid * NCHUNK, NCHUNK)], pidx)
        for n in range(NEG):
            pltpu.sync_copy(
                neg_h.at[pl.ds(n * (B // 128) + wid * NCHUNK, NCHUNK)],
                nidx.at[pl.ds(n * NCHUNK, NCHUNK)])

        def fire_ap(c):
            pltpu.async_copy(ew_h.at[aidx.at[c]], wa[c % 2], semA)
            pltpu.async_copy(ew_h.at[pidx.at[c]], wp[c % 2], semP)

        def fire_n(c, n):
            bi = (c * NEG + n) % 3
            pltpu.async_copy(ew_h.at[nidx.at[n * NCHUNK + c]],
                             wn[bi], semN[bi])

        def drain(buf, sem):
            pltpu.make_async_copy(ew_h.at[pl.ds(0, CHUNK)], buf, sem).wait()

        fire_ap(0)
        fire_n(0, 0)
        fire_n(0, 1)

        def dots(bufx, bufy, stage, soff):
            def group(g, carry):
                rowv = lane + g * 16

                def dbody(d, acc):
                    rot = (lane + d) & (D - 1)
                    x_d = plsc.load_gather(bufx, [rowv, rot])
                    y_d = plsc.load_gather(bufy, [rowv, rot])
                    return acc + x_d * y_d

                acc = lax.fori_loop(0, D, dbody,
                                    jnp.zeros((16,), jnp.float32))
                stage[pl.ds(soff + g * 16, 16)] = acc
                return carry

            lax.fori_loop(0, CHUNK // 16, group, 0)

        for c in range(NCHUNK):
            gc = wid * NCHUNK + c
            drain(wa[c % 2], semA)
            drain(wp[c % 2], semP)
            if c + 1 < NCHUNK:
                fire_ap(c + 1)
            dots(wa[c % 2], wp[c % 2], postage, 0)
            pltpu.sync_copy(postage, pos_out.at[pl.ds(gc * CHUNK, CHUNK)])
            for n in range(NEG):
                bi = (c * NEG + n) % 3
                drain(wn[bi], semN[bi])
                nf = n + 2
                if nf < NEG:
                    fire_n(c, nf)
                elif c + 1 < NCHUNK:
                    fire_n(c + 1, nf - NEG)
                dots(wa[c % 2], wn[bi], negstage, n * CHUNK)
            pltpu.sync_copy(
                negstage,
                neg_out.at[pl.ds(gc * CHUNK * NEG, CHUNK * NEG)])

    return k(ew, inp2d, pos2d, negT2d)


def _tc_loss(pos_s, neg_s):
    def body(p_ref, n_ref, o_ref):
        p = jax.nn.log_sigmoid(p_ref[...])
        n = jax.nn.log_sigmoid(n_ref[...])
        o_ref[...] = (jnp.sum(n) - jnp.sum(p))[None, None]

    out = pl.pallas_call(
        body,
        out_shape=jax.ShapeDtypeStruct((1, 1), jnp.float32),
    )(pos_s.reshape(B // 128, 128), neg_s.reshape(B * NEG // 128, 128))
    return out[0, 0]


def kernel(input, pos_input, neg_input, Embedding):
    inp = input.astype(jnp.int32).reshape(B // 128, 128)
    pos = pos_input.astype(jnp.int32).reshape(B // 128, 128)
    negT = neg_input.astype(jnp.int32).T.reshape(NEG * B // 128, 128)
    ew = _widen_table(Embedding)
    pos_s, neg_s = _sc_scores(ew, inp, pos, negT)
    return _tc_loss(pos_s, neg_s)


# speedup vs baseline: 2.8072x; 1.1204x over previous
import functools

import jax
import jax.numpy as jnp
from jax import lax
from jax.experimental import pallas as pl
from jax.experimental.pallas import tpu as pltpu
from jax.experimental.pallas import tpu_sc as plsc

B = 16384
V = 1000000
D = 64
NEG = 20
NC = 2
NS = 16
NW = NC * NS
RPW = B // NW
CHUNK = 128
NCHUNK = RPW // CHUNK
TBLK = 32768
NTB = (V + TBLK - 1) // TBLK
VPAD = NTB * TBLK


def _widen_table(emb):
    def body(x_ref, o_ref):
        t = x_ref[...].T
        o_ref[:, pl.ds(0, D)] = lax.slice(t, (0, 0), (TBLK // 2, D))
        o_ref[:, pl.ds(D, D)] = lax.slice(t, (TBLK // 2, 0), (TBLK, D))

    return pl.pallas_call(
        body,
        grid=(NTB,),
        in_specs=[pl.BlockSpec((D, TBLK), lambda g: (0, g))],
        out_specs=pl.BlockSpec((TBLK // 2, 128), lambda g: (g, 0)),
        out_shape=jax.ShapeDtypeStruct((VPAD // 2, 128), jnp.float32),
    )(emb.T)


def _sc_scores(ew, inp2d, pos2d, negT2d):
    mesh = plsc.VectorSubcoreMesh(core_axis_name="c", subcore_axis_name="s")

    @functools.partial(
        pl.kernel,
        mesh=mesh,
        compiler_params=pltpu.CompilerParams(
            needs_layout_passes=False, use_tc_tiling_on_sc=True),
        out_type=(
            jax.ShapeDtypeStruct((B,), jnp.float32),
            jax.ShapeDtypeStruct((B * NEG,), jnp.float32),
        ),
        scratch_types=[
            pltpu.VMEM((NCHUNK, 128), jnp.int32),
            pltpu.VMEM((NCHUNK, 128), jnp.int32),
            pltpu.VMEM((NEG * NCHUNK, 128), jnp.int32),
            pltpu.VMEM((CHUNK, 128), jnp.float32),
            pltpu.VMEM((CHUNK, 128), jnp.float32),
            pltpu.VMEM((CHUNK, 128), jnp.float32),
            pltpu.VMEM((CHUNK, 128), jnp.float32),
            pltpu.VMEM((CHUNK, 128), jnp.float32),
            pltpu.VMEM((CHUNK, 128), jnp.float32),
            pltpu.VMEM((CHUNK, 128), jnp.float32),
            pltpu.VMEM((8, 128), jnp.int32),
            pltpu.VMEM((CHUNK,), jnp.float32),
            pltpu.VMEM((CHUNK * NEG,), jnp.float32),
            pltpu.SemaphoreType.DMA,
            pltpu.SemaphoreType.DMA,
            pltpu.SemaphoreType.DMA,
            pltpu.SemaphoreType.DMA,
            pltpu.SemaphoreType.DMA,
        ],
    )
    def k(ew_h, inp_h, pos_h, neg_h, pos_out, neg_out,
          aidx, pidx, nidx, waA, waB, wpA, wpB, wnA, wnB, wnC,
          phys, postage, negstage, semA, semP, semN0, semN1, semN2):
        wid = lax.axis_index("s") * NC + lax.axis_index("c")
        lane = lax.iota(jnp.int32, 16)
        wa = (waA, waB)
        wp = (wpA, wpB)
        wn = (wnA, wnB, wnC)
        semN = (semN0, semN1, semN2)

        pltpu.sync_copy(inp_h.at[pl.ds(wid * NCHUNK, NCHUNK)], aidx)
        pltpu.sync_copy(pos_h.at[pl.ds(wid * NCHUNK, NCHUNK)], pidx)
        for n in range(NEG):
            pltpu.sync_copy(
                neg_h.at[pl.ds(n * (B // 128) + wid * NCHUNK, NCHUNK)],
                nidx.at[pl.ds(n * NCHUNK, NCHUNK)])

        HB = TBLK // 2
        def to_phys(src_ref, src_row, prow):
            for k in range(8):
                v = src_ref[src_row, pl.ds(k * 16, 16)]
                phys[prow, pl.ds(k * 16, 16)] = (
                    lax.shift_left(lax.shift_right_logical(v, 15), 14)
                    | (v & (HB - 1)))

        def fire_ap(c):
            to_phys(aidx, c, 3 + (c % 2))
            to_phys(pidx, c, 5 + (c % 2))
            pltpu.async_copy(ew_h.at[phys.at[3 + (c % 2)]], wa[c % 2], semA)
            pltpu.async_copy(ew_h.at[phys.at[5 + (c % 2)]], wp[c % 2], semP)

        def fire_n(c, n):
            bi = (c * NEG + n) % 3
            to_phys(nidx, n * NCHUNK + c, bi)
            pltpu.async_copy(ew_h.at[phys.at[bi]], wn[bi], semN[bi])

        def drain(buf, sem):
            pltpu.make_async_copy(ew_h.at[pl.ds(0, CHUNK)], buf, sem).wait()

        fire_ap(0)
        fire_n(0, 0)
        fire_n(0, 1)

        def dots(bufx, bufy, xrow, yrow, stage, soff):
            def group(g, carry):
                rowv = lane + g * 16
                hx = (aidx[xrow, pl.ds(g * 16, 16)] >> 8) & D
                hy_src, hy_row = yrow
                hy = (hy_src[hy_row, pl.ds(g * 16, 16)] >> 8) & D

                def dbody(d, acc):
                    rot = (lane + d) & (D - 1)
                    x_d = plsc.load_gather(bufx, [rowv, hx + rot])
                    y_d = plsc.load_gather(bufy, [rowv, hy + rot])
                    return acc + x_d * y_d

                acc = lax.fori_loop(0, D, dbody,
                                    jnp.zeros((16,), jnp.float32))
                stage[pl.ds(soff + g * 16, 16)] = acc
                return carry

            lax.fori_loop(0, CHUNK // 16, group, 0)

        for c in range(NCHUNK):
            gc = wid * NCHUNK + c
            drain(wa[c % 2], semA)
            drain(wp[c % 2], semP)
            if c + 1 < NCHUNK:
                fire_ap(c + 1)
            dots(wa[c % 2], wp[c % 2], c, (pidx, c), postage, 0)
            pltpu.sync_copy(postage, pos_out.at[pl.ds(gc * CHUNK, CHUNK)])
            for n in range(NEG):
                bi = (c * NEG + n) % 3
                drain(wn[bi], semN[bi])
                nf = n + 2
                if nf < NEG:
                    fire_n(c, nf)
                elif c + 1 < NCHUNK:
                    fire_n(c + 1, nf - NEG)
                dots(wa[c % 2], wn[bi], c, (nidx, n * NCHUNK + c),
                     negstage, n * CHUNK)
            pltpu.sync_copy(
                negstage,
                neg_out.at[pl.ds(gc * CHUNK * NEG, CHUNK * NEG)])

    return k(ew, inp2d, pos2d, negT2d)


def _tc_loss(pos_s, neg_s):
    def body(p_ref, n_ref, o_ref):
        p = jax.nn.log_sigmoid(p_ref[...])
        n = jax.nn.log_sigmoid(n_ref[...])
        o_ref[...] = (jnp.sum(n) - jnp.sum(p))[None, None]

    out = pl.pallas_call(
        body,
        out_shape=jax.ShapeDtypeStruct((1, 1), jnp.float32),
    )(pos_s.reshape(B // 128, 128), neg_s.reshape(B * NEG // 128, 128))
    return out[0, 0]


def kernel(input, pos_input, neg_input, Embedding):
    inp = input.astype(jnp.int32).reshape(B // 128, 128)
    pos = pos_input.astype(jnp.int32).reshape(B // 128, 128)
    negT = neg_input.astype(jnp.int32).T.reshape(NEG * B // 128, 128)
    ew = _widen_table(Embedding)
    pos_s, neg_s = _sc_scores(ew, inp, pos, negT)
    return _tc_loss(pos_s, neg_s)
